# baseline (speedup 1.0000x reference)
import functools

import jax
import jax.numpy as jnp
from jax import lax
from jax.experimental import pallas as pl
from jax.experimental.pallas import tpu as pltpu
from jax.experimental.pallas import tpu_sc as plsc

NG = 1000
D = 128
NC = 2
NS = 16
NW = NC * NS
CH = 128
NB = 3
NGP = 1024
RPT = NGP // 8


def _prep_edges(rows, cols, vals, col_offset):
    e = rows.shape[0]
    e_pad = -(-e // (NW * CH * NB)) * (NW * CH * NB)
    pad = e_pad - e
    rows = jnp.concatenate([rows.astype(jnp.int32), jnp.zeros((pad,), jnp.int32)])
    cols = jnp.concatenate([cols.astype(jnp.int32) + col_offset,
                            jnp.zeros((pad,), jnp.int32)])
    vals = jnp.concatenate([vals.astype(jnp.float32), jnp.zeros((pad,), jnp.float32)])
    cpt = e_pad // (NW * CH)
    shp = (NW, cpt, CH)
    return rows.reshape(shp), cols.reshape(shp), vals.reshape((NW, cpt * CH)), cpt


def _sc_segsum(table, ur, uc, uv, ir, ic, iv, cpt):
    mesh = plsc.VectorSubcoreMesh(core_axis_name="c", subcore_axis_name="s")

    @functools.partial(
        pl.kernel,
        out_type=jax.ShapeDtypeStruct((NC, 2, NGP, D), jnp.float32),
        mesh=mesh,
        scratch_types=[
            pltpu.VMEM((cpt, CH), jnp.int32),
            pltpu.VMEM((cpt, CH), jnp.int32),
            pltpu.VMEM((cpt * CH,), jnp.float32),
            pltpu.VMEM((3, CH, D), jnp.float32),
            pltpu.VMEM_SHARED((NGP, D), jnp.float32),
            pltpu.VMEM_SHARED((NGP, D), jnp.float32),
            pltpu.SemaphoreType.DMA((3,)),
        ],
    )
    def k(table_h, ur_h, uc_h, uv_h, ir_h, ic_h, iv_h, out_h,
          rows_v, cols_v, vals_v, gbufs, acc_u, acc_i, sg):
        cid = lax.axis_index("c")
        sid = lax.axis_index("s")
        wid = cid * NS + sid

        zero = jnp.zeros((16,), jnp.float32)
        zb = gbufs.at[0]

        def zrow(r, carry):
            for j in range(8):
                zb[r, pl.ds(16 * j, 16)] = zero
            return carry

        lax.fori_loop(0, CH, zrow, 0)

        @pl.when(sid < 8)
        def _():
            pltpu.sync_copy(zb, acc_u.at[pl.ds(sid * RPT, RPT)])

        @pl.when(sid >= 8)
        def _():
            pltpu.sync_copy(zb, acc_i.at[pl.ds((sid - 8) * RPT, RPT)])

        plsc.subcore_barrier()

        lane_idx = [jnp.full((16, 1), el, jnp.int32) for el in range(16)]
        gdn = lax.GatherDimensionNumbers(
            offset_dims=(), collapsed_slice_dims=(0,),
            start_index_map=(0,))

        def do_set(r_h, c_h, v_h, acc):
            pltpu.sync_copy(r_h.at[wid], rows_v)
            pltpu.sync_copy(c_h.at[wid], cols_v)
            pltpu.sync_copy(v_h.at[wid], vals_v)

            for b in range(2):
                pltpu.async_copy(table_h.at[cols_v.at[b]], gbufs.at[b],
                                 sg.at[b])

            def pair(k_, carry):
                for b in range(3):
                    c = 3 * k_ + b
                    gb = gbufs.at[b]
                    bn = (b + 2) % 3
                    @pl.when(c + 2 < cpt)
                    def _(bn=bn, c=c):
                        pltpu.async_copy(table_h.at[cols_v.at[c + 2]],
                                         gbufs.at[bn], sg.at[bn])

                    pltpu.make_async_copy(table_h.at[pl.ds(0, CH)], gb,
                                          sg.at[b]).wait()
                    cbase = c * CH

                    def group(g, gcarry, gb=gb, cbase=cbase):
                        vv = vals_v[pl.ds(cbase + g * 16, 16)]
                        row0 = g * 16
                        for el in range(16):
                            bvec = lax.gather(
                                vv, lane_idx[el], gdn, (1,),
                                mode=lax.GatherScatterMode.PROMISE_IN_BOUNDS)
                            for j in range(8):
                                sl = pl.ds(16 * j, 16)
                                gb[row0 + el, sl] = gb[row0 + el, sl] * bvec
                        return gcarry

                    lax.fori_loop(0, CH // 16, group, 0)
                    pltpu.sync_copy(gb, acc.at[rows_v.at[c]], add=True)
                return carry

            lax.fori_loop(0, cpt // 3, pair, 0)

        do_set(ur_h, uc_h, uv_h, acc_u)
        do_set(ir_h, ic_h, iv_h, acc_i)
        plsc.subcore_barrier()

        @pl.when(sid < 8)
        def _():
            s = sid * RPT
            pltpu.sync_copy(acc_u.at[pl.ds(s, RPT)],
                            out_h.at[cid, 0, pl.ds(s, RPT)])

        @pl.when(sid >= 8)
        def _():
            s = (sid - 8) * RPT
            pltpu.sync_copy(acc_i.at[pl.ds(s, RPT)],
                            out_h.at[cid, 1, pl.ds(s, RPT)])

    return k(table, ur, uc, uv, ir, ic, iv)


def _attn_body(m_ref, ge_ref, he_ref, w1_ref, b1_ref, w2_ref,
               uw_ref, ub_ref, iw_ref, ib_ref, msg_o, he_o):
    um = (m_ref[0, 0] + m_ref[1, 0])[:NG]
    im = (m_ref[0, 1] + m_ref[1, 1])[:NG]
    w1 = w1_ref[...]
    b1 = b1_ref[...][None, :]
    w2 = w2_ref[...][:, 0][None, :]
    hu = jnp.tanh(jnp.dot(um, w1, preferred_element_type=jnp.float32) + b1)
    hi = jnp.tanh(jnp.dot(im, w1, preferred_element_type=jnp.float32) + b1)
    qu = jnp.sum(hu * w2, axis=1, keepdims=True)
    qi = jnp.sum(hi * w2, axis=1, keepdims=True)
    mx = jnp.maximum(qu, qi)
    eu = jnp.exp(qu - mx)
    ei = jnp.exp(qi - mx)
    den = eu + ei
    common = (eu / den) * um + (ei / den) * im
    ge = ge_ref[...]
    uw, iw = uw_ref[...], iw_ref[...]
    um_o = (jnp.dot(um - common, uw[:D], preferred_element_type=jnp.float32)
            + jnp.dot(ge, uw[D:], preferred_element_type=jnp.float32)
            + ub_ref[...][None, :])
    im_o = (jnp.dot(im - common, iw[:D], preferred_element_type=jnp.float32)
            + jnp.dot(ge, iw[D:], preferred_element_type=jnp.float32)
            + ib_ref[...][None, :])
    msg = um_o + im_o + common
    msg_o[...] = msg
    he_o[...] = he_ref[...] + msg


def _attn(msgs, ge, he_in, w1, b1, w2, uw, ub, iw, ib):
    return pl.pallas_call(
        _attn_body,
        out_shape=(jax.ShapeDtypeStruct((NG, D), jnp.float32),
                   jax.ShapeDtypeStruct((NG, D), jnp.float32)),
    )(msgs, ge, he_in, w1, b1, w2, uw, ub, iw, ib)


_BM = 1000


def _mm_body(fh_ref, m_ref, out_ref):
    out_ref[...] = jnp.dot(fh_ref[...], m_ref[...],
                           preferred_element_type=jnp.float32)


def _mm(fh, msg):
    m = fh.shape[0]
    return pl.pallas_call(
        _mm_body,
        grid=(m // _BM,),
        in_specs=[
            pl.BlockSpec((_BM, NG), lambda i: (i, 0)),
            pl.BlockSpec((NG, D), lambda i: (0, 0)),
        ],
        out_specs=pl.BlockSpec((_BM, D), lambda i: (i, 0)),
        out_shape=jax.ShapeDtypeStruct((m, D), jnp.float32),
    )(fh, msg)


def _mm_final_body(fh_ref, m1_ref, m2_ref, base_ref, out_ref):
    out_ref[...] = base_ref[...] + jnp.dot(
        fh_ref[...], m1_ref[...] + m2_ref[...],
        preferred_element_type=jnp.float32)


def _mm_final(fh, msg1, msg2, base):
    m = fh.shape[0]
    return pl.pallas_call(
        _mm_final_body,
        grid=(m // _BM,),
        in_specs=[
            pl.BlockSpec((_BM, NG), lambda i: (i, 0)),
            pl.BlockSpec((NG, D), lambda i: (0, 0)),
            pl.BlockSpec((NG, D), lambda i: (0, 0)),
            pl.BlockSpec((_BM, D), lambda i: (i, 0)),
        ],
        out_specs=pl.BlockSpec((_BM, D), lambda i: (i, 0)),
        out_shape=jax.ShapeDtypeStruct((m, D), jnp.float32),
    )(fh, msg1, msg2, base)


def kernel(user_emb, item_emb, group_emb, num_users, num_items,
           u_rows, u_cols, u_vals, i_rows, i_cols, i_vals, full_hyper,
           qc_W1, qc_b1, qc_W2, ulin_W, ulin_b, ilin_W, ilin_b):
    nu = user_emb.shape[0]
    init_ui = jnp.concatenate([user_emb, item_emb], axis=0)

    ur, uc, uv, cpt = _prep_edges(u_rows, u_cols, u_vals, 0)
    ir, ic, iv, _ = _prep_edges(i_rows, i_cols, i_vals, nu)

    msgs1 = _sc_segsum(init_ui, ur, uc, uv, ir, ic, iv, cpt)
    msg1, he1 = _attn(msgs1, group_emb, group_emb, qc_W1[0], qc_b1[0],
                      qc_W2[0], ulin_W[0], ulin_b[0], ilin_W[0], ilin_b[0])
    node1 = _mm(full_hyper, msg1)

    msgs2 = _sc_segsum(node1, ur, uc, uv, ir, ic, iv, cpt)
    msg2, he2 = _attn(msgs2, group_emb, he1, qc_W1[1], qc_b1[1],
                      qc_W2[1], ulin_W[1], ulin_b[1], ilin_W[1], ilin_b[1])

    final_emb = _mm_final(full_hyper, msg1, msg2, init_ui)
    return jnp.concatenate([final_emb, he2], axis=0)


# speedup vs baseline: 1.7919x; 1.7919x over previous
import functools

import jax
import jax.numpy as jnp
from jax import lax
from jax.experimental import pallas as pl
from jax.experimental.pallas import tpu as pltpu
from jax.experimental.pallas import tpu_sc as plsc

NG = 1000
D = 128
NC = 2
NS = 16
NW = NC * NS
CH = 128
NB = 2
NGP = 1024
RPT = NGP // 8


def _prep_edges(rows, cols, vals, col_offset):
    e = rows.shape[0]
    e_pad = -(-e // (NW * CH * NB)) * (NW * CH * NB)
    pad = e_pad - e
    rows = jnp.concatenate([rows.astype(jnp.int32), jnp.zeros((pad,), jnp.int32)])
    cols = jnp.concatenate([cols.astype(jnp.int32) + col_offset,
                            jnp.zeros((pad,), jnp.int32)])
    vals = jnp.concatenate([vals.astype(jnp.float32), jnp.zeros((pad,), jnp.float32)])
    cpt = e_pad // (NW * CH)
    shp = (NW, cpt, CH)
    return rows.reshape(shp), cols.reshape(shp), vals.reshape((NW, cpt * CH)), cpt


def _sc_segsum(table, ur, uc, uv, ir, ic, iv, cpt):
    mesh = plsc.VectorSubcoreMesh(core_axis_name="c", subcore_axis_name="s")

    @functools.partial(
        pl.kernel,
        out_type=jax.ShapeDtypeStruct((NC, 2, NGP, D), jnp.float32),
        mesh=mesh,
        scratch_types=[
            pltpu.VMEM((cpt, CH), jnp.int32),
            pltpu.VMEM((cpt, CH), jnp.int32),
            pltpu.VMEM((cpt * CH,), jnp.float32),
            pltpu.VMEM((2, CH, D), jnp.float32),
            pltpu.VMEM_SHARED((NGP, D), jnp.float32),
            pltpu.VMEM_SHARED((NGP, D), jnp.float32),
            pltpu.SemaphoreType.DMA((2,)),
        ],
    )
    def k(table_h, ur_h, uc_h, uv_h, ir_h, ic_h, iv_h, out_h,
          rows_v, cols_v, vals_v, gbufs, acc_u, acc_i, sg):
        cid = lax.axis_index("c")
        sid = lax.axis_index("s")
        wid = cid * NS + sid

        zero = jnp.zeros((16,), jnp.float32)
        zb = gbufs.at[0]

        def zrow(r, carry):
            for j in range(8):
                zb[r, pl.ds(16 * j, 16)] = zero
            return carry

        lax.fori_loop(0, CH, zrow, 0)

        @pl.when(sid < 8)
        def _():
            pltpu.sync_copy(zb, acc_u.at[pl.ds(sid * RPT, RPT)])

        @pl.when(sid >= 8)
        def _():
            pltpu.sync_copy(zb, acc_i.at[pl.ds((sid - 8) * RPT, RPT)])

        plsc.subcore_barrier()

        lane_idx = [jnp.full((16, 1), el, jnp.int32) for el in range(16)]
        gdn = lax.GatherDimensionNumbers(
            offset_dims=(), collapsed_slice_dims=(0,),
            start_index_map=(0,))

        def do_set(r_h, c_h, v_h, acc):
            pltpu.sync_copy(r_h.at[wid], rows_v)
            pltpu.sync_copy(c_h.at[wid], cols_v)
            pltpu.sync_copy(v_h.at[wid], vals_v)

            pltpu.async_copy(table_h.at[cols_v.at[0]], gbufs.at[0], sg.at[0])

            def pair(k_, carry):
                for b in range(2):
                    c = 2 * k_ + b
                    gb = gbufs.at[b]
                    bn = 1 - b
                    @pl.when(c + 1 < cpt)
                    def _(bn=bn, c=c):
                        pltpu.async_copy(table_h.at[cols_v.at[c + 1]],
                                         gbufs.at[bn], sg.at[bn])

                    pltpu.make_async_copy(table_h.at[pl.ds(0, CH)], gb,
                                          sg.at[b]).wait()
                    cbase = c * CH

                    def group(g, gcarry, gb=gb, cbase=cbase):
                        vv = vals_v[pl.ds(cbase + g * 16, 16)]
                        row0 = g * 16
                        for el in range(16):
                            bvec = lax.gather(
                                vv, lane_idx[el], gdn, (1,),
                                mode=lax.GatherScatterMode.PROMISE_IN_BOUNDS)
                            for j in range(8):
                                sl = pl.ds(16 * j, 16)
                                gb[row0 + el, sl] = gb[row0 + el, sl] * bvec
                        return gcarry

                    lax.fori_loop(0, CH // 16, group, 0)
                    pltpu.sync_copy(gb, acc.at[rows_v.at[c]], add=True)
                return carry

            lax.fori_loop(0, cpt // 2, pair, 0)

        do_set(ur_h, uc_h, uv_h, acc_u)
        do_set(ir_h, ic_h, iv_h, acc_i)
        plsc.subcore_barrier()

        @pl.when(sid < 8)
        def _():
            s = sid * RPT
            pltpu.sync_copy(acc_u.at[pl.ds(s, RPT)],
                            out_h.at[cid, 0, pl.ds(s, RPT)])

        @pl.when(sid >= 8)
        def _():
            s = (sid - 8) * RPT
            pltpu.sync_copy(acc_i.at[pl.ds(s, RPT)],
                            out_h.at[cid, 1, pl.ds(s, RPT)])

    return k(table, ur, uc, uv, ir, ic, iv)


def _attn_body(m_ref, ge_ref, he_ref, w1_ref, b1_ref, w2_ref,
               uw_ref, ub_ref, iw_ref, ib_ref, msg_o, he_o):
    um = (m_ref[0, 0] + m_ref[1, 0])[:NG]
    im = (m_ref[0, 1] + m_ref[1, 1])[:NG]
    w1 = w1_ref[...]
    b1 = b1_ref[...][None, :]
    w2 = w2_ref[...][:, 0][None, :]
    hu = jnp.tanh(jnp.dot(um, w1, preferred_element_type=jnp.float32) + b1)
    hi = jnp.tanh(jnp.dot(im, w1, preferred_element_type=jnp.float32) + b1)
    qu = jnp.sum(hu * w2, axis=1, keepdims=True)
    qi = jnp.sum(hi * w2, axis=1, keepdims=True)
    mx = jnp.maximum(qu, qi)
    eu = jnp.exp(qu - mx)
    ei = jnp.exp(qi - mx)
    den = eu + ei
    common = (eu / den) * um + (ei / den) * im
    ge = ge_ref[...]
    uw, iw = uw_ref[...], iw_ref[...]
    um_o = (jnp.dot(um - common, uw[:D], preferred_element_type=jnp.float32)
            + jnp.dot(ge, uw[D:], preferred_element_type=jnp.float32)
            + ub_ref[...][None, :])
    im_o = (jnp.dot(im - common, iw[:D], preferred_element_type=jnp.float32)
            + jnp.dot(ge, iw[D:], preferred_element_type=jnp.float32)
            + ib_ref[...][None, :])
    msg = um_o + im_o + common
    msg_o[...] = msg
    he_o[...] = he_ref[...] + msg


def _attn(msgs, ge, he_in, w1, b1, w2, uw, ub, iw, ib):
    return pl.pallas_call(
        _attn_body,
        out_shape=(jax.ShapeDtypeStruct((NG, D), jnp.float32),
                   jax.ShapeDtypeStruct((NG, D), jnp.float32)),
    )(msgs, ge, he_in, w1, b1, w2, uw, ub, iw, ib)


_BM = 1000


def _mm_body(fh_ref, m_ref, out_ref):
    out_ref[...] = jnp.dot(fh_ref[...], m_ref[...],
                           preferred_element_type=jnp.float32)


def _mm(fh, msg):
    m = fh.shape[0]
    return pl.pallas_call(
        _mm_body,
        grid=(m // _BM,),
        in_specs=[
            pl.BlockSpec((_BM, NG), lambda i: (i, 0)),
            pl.BlockSpec((NG, D), lambda i: (0, 0)),
        ],
        out_specs=pl.BlockSpec((_BM, D), lambda i: (i, 0)),
        out_shape=jax.ShapeDtypeStruct((m, D), jnp.float32),
    )(fh, msg)


def _mm_final_body(fh_ref, m1_ref, m2_ref, base_ref, out_ref):
    out_ref[...] = base_ref[...] + jnp.dot(
        fh_ref[...], m1_ref[...] + m2_ref[...],
        preferred_element_type=jnp.float32)


def _mm_final(fh, msg1, msg2, base):
    m = fh.shape[0]
    return pl.pallas_call(
        _mm_final_body,
        grid=(m // _BM,),
        in_specs=[
            pl.BlockSpec((_BM, NG), lambda i: (i, 0)),
            pl.BlockSpec((NG, D), lambda i: (0, 0)),
            pl.BlockSpec((NG, D), lambda i: (0, 0)),
            pl.BlockSpec((_BM, D), lambda i: (i, 0)),
        ],
        out_specs=pl.BlockSpec((_BM, D), lambda i: (i, 0)),
        out_shape=jax.ShapeDtypeStruct((m, D), jnp.float32),
    )(fh, msg1, msg2, base)


def kernel(user_emb, item_emb, group_emb, num_users, num_items,
           u_rows, u_cols, u_vals, i_rows, i_cols, i_vals, full_hyper,
           qc_W1, qc_b1, qc_W2, ulin_W, ulin_b, ilin_W, ilin_b):
    nu = user_emb.shape[0]
    init_ui = jnp.concatenate([user_emb, item_emb], axis=0)

    ur, uc, uv, cpt = _prep_edges(u_rows, u_cols, u_vals, 0)
    ir, ic, iv, _ = _prep_edges(i_rows, i_cols, i_vals, nu)

    msgs1 = _sc_segsum(init_ui, ur, uc, uv, ir, ic, iv, cpt)
    msg1, he1 = _attn(msgs1, group_emb, group_emb, qc_W1[0], qc_b1[0],
                      qc_W2[0], ulin_W[0], ulin_b[0], ilin_W[0], ilin_b[0])
    node1 = _mm(full_hyper, msg1)

    msgs2 = _sc_segsum(node1, ur, uc, uv, ir, ic, iv, cpt)
    msg2, he2 = _attn(msgs2, group_emb, he1, qc_W1[1], qc_b1[1],
                      qc_W2[1], ulin_W[1], ulin_b[1], ilin_W[1], ilin_b[1])

    final_emb = _mm_final(full_hyper, msg1, msg2, init_ui)
    return jnp.concatenate([final_emb, he2], axis=0)
